# baseline (speedup 1.0000x reference)
import functools

import jax
import jax.numpy as jnp
from jax import lax
from jax.experimental import pallas as pl
from jax.experimental.pallas import tpu as pltpu
from jax.experimental.pallas import tpu_sc as plsc

N_ENT = 9999
N_REL = 237
H = 128
NB = 4
NBATCH = 3333
E = 9999 * 16

NW = 32
EDGES_PER_W = 5120
E_PAD = NW * EDGES_PER_W
CHUNK = 32
N_CHUNKS = EDGES_PER_W // CHUNK
SUP = 512
N_SUP = EDGES_PER_W // SUP
GC = 16
NQUAD = SUP // (4 * GC)
DCHUNK = 64
DNPAIR = EDGES_PER_W // (2 * DCHUNK)
ACC_ROWS = 10112
STRIPE = ACC_ROWS // 16
CREP_W = NB * 16


def _bf16_bits(x):
    u = lax.bitcast_convert_type(x, jnp.int32)
    return u + 32767 + ((u >> 16) & 1)


def _pack_i32(lo, hi):
    return (_bf16_bits(hi) & jnp.int32(-65536)) | (
        (_bf16_bits(lo) >> 16) & 65535)


def _gru_body(x_ref, h0_ref, wi0, wh0, bi0, bh0, wi1, wh1, bi1, bh1, vlo, vhi,
              out_ref):
    h1 = h0_ref[0]
    h2 = h0_ref[1]
    for t in range(3):
        xt = x_ref[t]
        gi = jnp.dot(xt, wi0[:], preferred_element_type=jnp.float32) + bi0[:]
        gh = jnp.dot(h1, wh0[:], preferred_element_type=jnp.float32) + bh0[:]
        r = jax.nn.sigmoid(gi[:, :H] + gh[:, :H])
        z = jax.nn.sigmoid(gi[:, H:2 * H] + gh[:, H:2 * H])
        n = jnp.tanh(gi[:, 2 * H:] + r * gh[:, 2 * H:])
        h1 = (1.0 - z) * n + z * h1
        gi = jnp.dot(h1, wi1[:], preferred_element_type=jnp.float32) + bi1[:]
        gh = jnp.dot(h2, wh1[:], preferred_element_type=jnp.float32) + bh1[:]
        r = jax.nn.sigmoid(gi[:, :H] + gh[:, :H])
        z = jax.nn.sigmoid(gi[:, H:2 * H] + gh[:, H:2 * H])
        n = jnp.tanh(gi[:, 2 * H:] + r * gh[:, 2 * H:])
        h2 = (1.0 - z) * n + z * h2
        lo = jnp.dot(h2, vlo[:], preferred_element_type=jnp.float32)
        hi = jnp.dot(h2, vhi[:], preferred_element_type=jnp.float32)
        out_ref[t] = _pack_i32(lo, hi)


def _gru_basis(x_stack, h0, wi0, wh0, bi0, bh0, wi1, wh1, bi1, bh1, vlo, vhi):
    grid = (pl.cdiv(NBATCH, H),)
    full = lambda shape: pl.BlockSpec(shape, lambda g: tuple(0 for _ in shape))
    w = NB * H // 2
    return pl.pallas_call(
        _gru_body,
        grid=grid,
        in_specs=[
            pl.BlockSpec((3, H, H), lambda g: (0, g, 0)),
            pl.BlockSpec((2, H, H), lambda g: (0, g, 0)),
            full((H, 3 * H)), full((H, 3 * H)), full((1, 3 * H)),
            full((1, 3 * H)), full((H, 3 * H)), full((H, 3 * H)),
            full((1, 3 * H)), full((1, 3 * H)), full((H, w)), full((H, w)),
        ],
        out_specs=pl.BlockSpec((3, H, w), lambda g: (0, g, 0)),
        out_shape=jax.ShapeDtypeStruct((3, NBATCH, w), jnp.int32),
    )(x_stack, h0, wi0, wh0, bi0, bh0, wi1, wh1, bi1, bh1, vlo, vhi)


def _deg_body(dst_hbm, z128_hbm, deg_out,
              deg_sh, dst_a, dst_b, ones_v, dsem0, dsem1):
    c = lax.axis_index("c")
    s = lax.axis_index("s")
    wid = s * 2 + c
    pltpu.sync_copy(z128_hbm, deg_sh.at[pl.ds(s * STRIPE, STRIPE)])

    def fill_ones(i, carry):
        for j in range(H // 16):
            ones_v[i, pl.ds(j * 16, 16)] = jnp.full((16,), 1.0, jnp.float32)
        return carry
    lax.fori_loop(0, DCHUNK, fill_ones, 0)
    plsc.subcore_barrier()

    base = wid * EDGES_PER_W
    pltpu.async_copy(dst_hbm.at[pl.ds(base, DCHUNK)], dst_a, dsem0)

    def pair_body(p, carry):
        off = base + p * 2 * DCHUNK
        d1 = pltpu.async_copy(dst_hbm.at[pl.ds(off + DCHUNK, DCHUNK)],
                              dst_b, dsem1)
        pltpu.make_async_copy(dst_hbm.at[pl.ds(off, DCHUNK)],
                              dst_a, dsem0).wait()
        pltpu.sync_copy(ones_v, deg_sh.at[dst_a], add=True)

        @pl.when(p < DNPAIR - 1)
        def _():
            pltpu.async_copy(dst_hbm.at[pl.ds(off + 2 * DCHUNK, DCHUNK)],
                             dst_a, dsem0)

        d1.wait()
        pltpu.sync_copy(ones_v, deg_sh.at[dst_b], add=True)
        return carry

    lax.fori_loop(0, DNPAIR, pair_body, 0)
    plsc.subcore_barrier()
    pltpu.sync_copy(deg_sh.at[pl.ds(s * STRIPE, STRIPE)],
                    deg_out.at[c, pl.ds(s * STRIPE, STRIPE)])


def _deg_pass(dst_p, z128):
    mesh = plsc.VectorSubcoreMesh(core_axis_name="c", subcore_axis_name="s")
    fn = pl.kernel(
        _deg_body,
        mesh=mesh,
        out_type=jax.ShapeDtypeStruct((2, ACC_ROWS, H), jnp.float32),
        scratch_types=[
            pltpu.VMEM_SHARED((ACC_ROWS, H), jnp.float32),
            pltpu.VMEM((DCHUNK,), jnp.int32),
            pltpu.VMEM((DCHUNK,), jnp.int32),
            pltpu.VMEM((DCHUNK, H), jnp.float32),
            pltpu.SemaphoreType.DMA,
            pltpu.SemaphoreType.DMA,
        ],
    )
    return fn(dst_p, z128)


def _mix_chunk(rows_b, row_off, rel_sv, crep_v, msg_v, co):
    relg = rel_sv[pl.ds(co, 16)]
    for k0 in range(16):
        k = row_off + k0
        rel_s = relg[k0]
        cbase = rel_s * CREP_W
        ce = [crep_v[pl.ds(cbase + b * 16, 16)] for b in range(NB)]
        for g2 in range(H // 32):
            def halves(b):
                off = b * 64 + g2 * 16
                w = rows_b[k, off // 128, pl.ds(off % 128, 16)]
                ev = lax.bitcast_convert_type(w << 16, jnp.float32)
                od = lax.bitcast_convert_type(w & jnp.int32(-65536),
                                              jnp.float32)
                return ev, od

            a0, b0 = halves(0)
            m0 = ce[0] * a0
            m1 = ce[0] * b0
            for b in range(1, NB):
                ab, bb = halves(b)
                m0 = m0 + ce[b] * ab
                m1 = m1 + ce[b] * bb
            msg_v[k0, pl.ds(g2 * 32, 16)] = m0
            msg_v[k0, pl.ds(g2 * 32 + 16, 16)] = m1


def _edge_body(t_hbm, crep_hbm, src_hbm, dst_hbm, rel_hbm, z128_hbm,
               acc_out,
               acc_sh, src_sv, dst_sv, rel_sv, rows0, rows1, rows2, rows3,
               msg0, msg1, crep_v, gsem0, gsem1, gsem2, gsem3, asem0, asem1):
    c = lax.axis_index("c")
    s = lax.axis_index("s")
    wid = s * 2 + c

    pltpu.sync_copy(z128_hbm, acc_sh.at[pl.ds(s * STRIPE, STRIPE)])
    pltpu.sync_copy(crep_hbm, crep_v)
    plsc.subcore_barrier()

    base = wid * EDGES_PER_W

    def gather(co, buf, sem):
        return pltpu.async_copy(t_hbm.at[src_sv[pl.ds(co, 16)]], buf, sem)

    def gather_wait(buf, sem):
        pltpu.make_async_copy(t_hbm.at[src_sv[pl.ds(0, 16)]],
                              buf, sem).wait()

    def scatter(msg_b, co, sem):
        pltpu.async_copy(msg_b, acc_sh.at[dst_sv[pl.ds(co, 16)]], sem,
                         add=True)

    def scatter_wait(msg_b, sem):
        pltpu.make_async_copy(msg_b, acc_sh.at[dst_sv[pl.ds(0, 16)]],
                              sem).wait()

    rows = [rows0, rows1, rows2, rows3]
    gsems = [gsem0, gsem1, gsem2, gsem3]
    msgs = [msg0, msg1]
    asems = [asem0, asem1]

    def sup_body(sc, carry):
        soff = base + sc * SUP
        pltpu.sync_copy(src_hbm.at[pl.ds(soff, SUP)], src_sv)
        pltpu.sync_copy(dst_hbm.at[pl.ds(soff, SUP)], dst_sv)
        pltpu.sync_copy(rel_hbm.at[pl.ds(soff, SUP)], rel_sv)
        gather(0, rows[0], gsems[0])
        gather(16, rows[1], gsems[1])

        def quad_body(q, carry2):
            for j in range(4):
                co = q * 64 + j * 16
                first = jnp.logical_and(sc == 0, q == 0) if j < 2 else None
                bj = (j + 2) % 4
                if j < 2:
                    gather(co + 32, rows[bj], gsems[bj])
                else:
                    @pl.when(q < NQUAD - 1)
                    def _():
                        gather(co + 32, rows[bj], gsems[bj])
                gather_wait(rows[j], gsems[j])
                mb, ab = msgs[j % 2], asems[j % 2]
                if j < 2:
                    @pl.when(jnp.logical_not(first))
                    def _():
                        scatter_wait(mb, ab)
                else:
                    scatter_wait(mb, ab)
                _mix_chunk(rows[j], 0, rel_sv, crep_v, mb, co)
                scatter(mb, co, ab)
            return carry2

        lax.fori_loop(0, NQUAD, quad_body, 0)
        return carry

    lax.fori_loop(0, N_SUP, sup_body, 0)
    scatter_wait(msg0, asem0)
    scatter_wait(msg1, asem1)
    plsc.subcore_barrier()

    pltpu.sync_copy(acc_sh.at[pl.ds(s * STRIPE, STRIPE)],
                    acc_out.at[c, pl.ds(s * STRIPE, STRIPE)])


def _edge_pass(t_flat, crep, src_p, dst_p, rel_p, z128):
    mesh = plsc.VectorSubcoreMesh(core_axis_name="c", subcore_axis_name="s")
    fn = pl.kernel(
        _edge_body,
        mesh=mesh,
        out_type=jax.ShapeDtypeStruct((2, ACC_ROWS, H), jnp.float32),
        scratch_types=[
            pltpu.VMEM_SHARED((ACC_ROWS, H), jnp.float32),
            pltpu.VMEM((SUP,), jnp.int32),
            pltpu.VMEM((SUP,), jnp.int32),
            pltpu.VMEM((SUP,), jnp.int32),
            pltpu.VMEM((GC, 2, H), jnp.int32),
            pltpu.VMEM((GC, 2, H), jnp.int32),
            pltpu.VMEM((GC, 2, H), jnp.int32),
            pltpu.VMEM((GC, 2, H), jnp.int32),
            pltpu.VMEM((16, H), jnp.float32),
            pltpu.VMEM((16, H), jnp.float32),
            pltpu.VMEM((240 * CREP_W,), jnp.float32),
            pltpu.SemaphoreType.DMA,
            pltpu.SemaphoreType.DMA,
            pltpu.SemaphoreType.DMA,
            pltpu.SemaphoreType.DMA,
            pltpu.SemaphoreType.DMA,
            pltpu.SemaphoreType.DMA,
        ],
    )
    return fn(t_flat, crep, src_p, dst_p, rel_p, z128)


def _combine_basis_body(acc_ref, deg_ref, bias_ref, vlo_ref, vhi_ref,
                        out_ref):
    a = acc_ref[0] + acc_ref[1]
    d = deg_ref[0, :, 0:1] + deg_ref[1, :, 0:1]
    norm = jnp.where(d > 0, 1.0 / jnp.maximum(d, 1.0), 0.0)
    h = a * norm + bias_ref[:]
    lo = jnp.dot(h, vlo_ref[:], preferred_element_type=jnp.float32)
    hi = jnp.dot(h, vhi_ref[:], preferred_element_type=jnp.float32)
    out_ref[...] = _pack_i32(lo, hi)


def _combine_basis(acc, deg, bias, vlo, vhi):
    grid = (pl.cdiv(N_ENT, H),)
    w = NB * H // 2
    return pl.pallas_call(
        _combine_basis_body,
        grid=grid,
        in_specs=[
            pl.BlockSpec((2, H, H), lambda g: (0, g, 0)),
            pl.BlockSpec((2, H, H), lambda g: (0, g, 0)),
            pl.BlockSpec((1, H), lambda g: (0, 0)),
            pl.BlockSpec((H, w), lambda g: (0, 0)),
            pl.BlockSpec((H, w), lambda g: (0, 0)),
        ],
        out_specs=pl.BlockSpec((H, w), lambda g: (g, 0)),
        out_shape=jax.ShapeDtypeStruct((N_ENT, w), jnp.int32),
    )(acc, deg, bias, vlo, vhi)


def _final_body(acc_ref, deg_ref, bias_ref, out_ref):
    a = acc_ref[0] + acc_ref[1]
    d = deg_ref[0, :, 0:1] + deg_ref[1, :, 0:1]
    norm = jnp.where(d > 0, 1.0 / jnp.maximum(d, 1.0), 0.0)
    out_ref[...] = jnp.maximum(a * norm + bias_ref[:], 0.0)


def _final(acc, deg, bias):
    grid = (pl.cdiv(N_ENT, H),)
    return pl.pallas_call(
        _final_body,
        grid=grid,
        in_specs=[
            pl.BlockSpec((2, H, H), lambda g: (0, g, 0)),
            pl.BlockSpec((2, H, H), lambda g: (0, g, 0)),
            pl.BlockSpec((1, H), lambda g: (0, 0)),
        ],
        out_specs=pl.BlockSpec((H, H), lambda g: (g, 0)),
        out_shape=jax.ShapeDtypeStruct((N_ENT, H), jnp.float32),
    )(acc, deg, bias)


def kernel(ent_emb, rel_emb, V1, coeff1, bias1, V2, coeff2, bias2,
           W_ih0, W_hh0, b_ih0, b_hh0, W_ih1, W_hh1, b_ih1, b_hh1,
           h0, edge_index, rel_id):
    del rel_emb

    x_stack = ent_emb.reshape(3, NBATCH, H)
    def vsplit(v):
        vc = v.transpose(1, 0, 2).reshape(H, NB, NB, 2, 16)
        return (vc[:, :, :, 0, :].reshape(H, NB * H // 2),
                vc[:, :, :, 1, :].reshape(H, NB * H // 2))
    v1lo, v1hi = vsplit(V1)
    v2lo, v2hi = vsplit(V2)
    wi0 = W_ih0.T
    wh0 = W_hh0.T
    wi1 = W_ih1.T
    wh1 = W_hh1.T
    bi0 = b_ih0.reshape(1, 3 * H)
    bh0 = b_hh0.reshape(1, 3 * H)
    bi1 = b_ih1.reshape(1, 3 * H)
    bh1 = b_hh1.reshape(1, 3 * H)

    pad = E_PAD - E
    src_p = jnp.concatenate([edge_index[0], jnp.zeros((pad,), jnp.int32)])
    dst_p = jnp.concatenate(
        [edge_index[1], jnp.full((pad,), ACC_ROWS - 1, jnp.int32)])
    rel_p = jnp.concatenate([rel_id, jnp.zeros((pad,), jnp.int32)])
    z128 = jnp.zeros((STRIPE, H), jnp.float32)
    crep1 = jnp.pad(jnp.repeat(coeff1, 16, axis=1),
                    ((0, 240 - N_REL), (0, 0))).reshape(-1)
    crep2 = jnp.pad(jnp.repeat(coeff2, 16, axis=1),
                    ((0, 240 - N_REL), (0, 0))).reshape(-1)

    deg = _deg_pass(dst_p, z128)
    t1 = _gru_basis(x_stack, h0, wi0, wh0, bi0, bh0, wi1, wh1,
                    bi1, bh1, v1lo, v1hi).reshape(N_ENT, 2, H)

    acc1 = _edge_pass(t1, crep1, src_p, dst_p, rel_p, z128)

    t2 = _combine_basis(acc1, deg, bias1.reshape(1, H), v2lo, v2hi
                        ).reshape(N_ENT, 2, H)

    acc2 = _edge_pass(t2, crep2, src_p, dst_p, rel_p, z128)

    return _final(acc2, deg, bias2.reshape(1, H))


# speedup vs baseline: 1.0769x; 1.0769x over previous
import functools

import jax
import jax.numpy as jnp
from jax import lax
from jax.experimental import pallas as pl
from jax.experimental.pallas import tpu as pltpu
from jax.experimental.pallas import tpu_sc as plsc

N_ENT = 9999
N_REL = 237
H = 128
NB = 4
NBATCH = 3333
E = 9999 * 16

NW = 32
EDGES_PER_W = 5120
E_PAD = NW * EDGES_PER_W
CHUNK = 32
N_CHUNKS = EDGES_PER_W // CHUNK
SUP = 512
N_SUP = EDGES_PER_W // SUP
GC = 16
NPAIR = SUP // (2 * GC)
DCHUNK = 64
DNPAIR = EDGES_PER_W // (2 * DCHUNK)
ACC_ROWS = 10112
STRIPE = ACC_ROWS // 16
CREP_W = NB * 16


def _bf16_bits(x):
    u = lax.bitcast_convert_type(x, jnp.int32)
    return u + 32767 + ((u >> 16) & 1)


def _pack_i32(lo, hi):
    return (_bf16_bits(hi) & jnp.int32(-65536)) | (
        (_bf16_bits(lo) >> 16) & 65535)


def _gru_body(x_ref, h0_ref, wi0, wh0, bi0, bh0, wi1, wh1, bi1, bh1, vlo, vhi,
              out_ref):
    h1 = h0_ref[0]
    h2 = h0_ref[1]
    for t in range(3):
        xt = x_ref[t]
        gi = jnp.dot(xt, wi0[:], preferred_element_type=jnp.float32) + bi0[:]
        gh = jnp.dot(h1, wh0[:], preferred_element_type=jnp.float32) + bh0[:]
        r = jax.nn.sigmoid(gi[:, :H] + gh[:, :H])
        z = jax.nn.sigmoid(gi[:, H:2 * H] + gh[:, H:2 * H])
        n = jnp.tanh(gi[:, 2 * H:] + r * gh[:, 2 * H:])
        h1 = (1.0 - z) * n + z * h1
        gi = jnp.dot(h1, wi1[:], preferred_element_type=jnp.float32) + bi1[:]
        gh = jnp.dot(h2, wh1[:], preferred_element_type=jnp.float32) + bh1[:]
        r = jax.nn.sigmoid(gi[:, :H] + gh[:, :H])
        z = jax.nn.sigmoid(gi[:, H:2 * H] + gh[:, H:2 * H])
        n = jnp.tanh(gi[:, 2 * H:] + r * gh[:, 2 * H:])
        h2 = (1.0 - z) * n + z * h2
        lo = jnp.dot(h2, vlo[:], preferred_element_type=jnp.float32)
        hi = jnp.dot(h2, vhi[:], preferred_element_type=jnp.float32)
        out_ref[t] = _pack_i32(lo, hi)


def _gru_basis(x_stack, h0, wi0, wh0, bi0, bh0, wi1, wh1, bi1, bh1, vlo, vhi):
    grid = (pl.cdiv(NBATCH, H),)
    full = lambda shape: pl.BlockSpec(shape, lambda g: tuple(0 for _ in shape))
    w = NB * H // 2
    return pl.pallas_call(
        _gru_body,
        grid=grid,
        in_specs=[
            pl.BlockSpec((3, H, H), lambda g: (0, g, 0)),
            pl.BlockSpec((2, H, H), lambda g: (0, g, 0)),
            full((H, 3 * H)), full((H, 3 * H)), full((1, 3 * H)),
            full((1, 3 * H)), full((H, 3 * H)), full((H, 3 * H)),
            full((1, 3 * H)), full((1, 3 * H)), full((H, w)), full((H, w)),
        ],
        out_specs=pl.BlockSpec((3, H, w), lambda g: (0, g, 0)),
        out_shape=jax.ShapeDtypeStruct((3, NBATCH, w), jnp.int32),
    )(x_stack, h0, wi0, wh0, bi0, bh0, wi1, wh1, bi1, bh1, vlo, vhi)


def _deg_body(dst_hbm, z128_hbm, deg_out,
              deg_sh, dst_a, dst_b, ones_v, dsem0, dsem1):
    c = lax.axis_index("c")
    s = lax.axis_index("s")
    wid = s * 2 + c
    pltpu.sync_copy(z128_hbm, deg_sh.at[pl.ds(s * STRIPE, STRIPE)])

    def fill_ones(i, carry):
        for j in range(H // 16):
            ones_v[i, pl.ds(j * 16, 16)] = jnp.full((16,), 1.0, jnp.float32)
        return carry
    lax.fori_loop(0, DCHUNK, fill_ones, 0)
    plsc.subcore_barrier()

    base = wid * EDGES_PER_W
    pltpu.async_copy(dst_hbm.at[pl.ds(base, DCHUNK)], dst_a, dsem0)

    def pair_body(p, carry):
        off = base + p * 2 * DCHUNK
        d1 = pltpu.async_copy(dst_hbm.at[pl.ds(off + DCHUNK, DCHUNK)],
                              dst_b, dsem1)
        pltpu.make_async_copy(dst_hbm.at[pl.ds(off, DCHUNK)],
                              dst_a, dsem0).wait()
        pltpu.sync_copy(ones_v, deg_sh.at[dst_a], add=True)

        @pl.when(p < DNPAIR - 1)
        def _():
            pltpu.async_copy(dst_hbm.at[pl.ds(off + 2 * DCHUNK, DCHUNK)],
                             dst_a, dsem0)

        d1.wait()
        pltpu.sync_copy(ones_v, deg_sh.at[dst_b], add=True)
        return carry

    lax.fori_loop(0, DNPAIR, pair_body, 0)
    plsc.subcore_barrier()
    pltpu.sync_copy(deg_sh.at[pl.ds(s * STRIPE, STRIPE)],
                    deg_out.at[c, pl.ds(s * STRIPE, STRIPE)])


def _deg_pass(dst_p, z128):
    mesh = plsc.VectorSubcoreMesh(core_axis_name="c", subcore_axis_name="s")
    fn = pl.kernel(
        _deg_body,
        mesh=mesh,
        out_type=jax.ShapeDtypeStruct((2, ACC_ROWS, H), jnp.float32),
        scratch_types=[
            pltpu.VMEM_SHARED((ACC_ROWS, H), jnp.float32),
            pltpu.VMEM((DCHUNK,), jnp.int32),
            pltpu.VMEM((DCHUNK,), jnp.int32),
            pltpu.VMEM((DCHUNK, H), jnp.float32),
            pltpu.SemaphoreType.DMA,
            pltpu.SemaphoreType.DMA,
        ],
    )
    return fn(dst_p, z128)


def _mix_chunk(rows_b, row_off, rel_sv, crep_v, msg_v, co):
    relg = rel_sv[pl.ds(co, 16)]
    for k0 in range(16):
        k = row_off + k0
        rel_s = relg[k0]
        cbase = rel_s * CREP_W
        ce = [crep_v[pl.ds(cbase + b * 16, 16)] for b in range(NB)]
        for g2 in range(H // 32):
            def halves(b):
                off = b * 64 + g2 * 16
                w = rows_b[k, off // 128, pl.ds(off % 128, 16)]
                ev = lax.bitcast_convert_type(w << 16, jnp.float32)
                od = lax.bitcast_convert_type(w & jnp.int32(-65536),
                                              jnp.float32)
                return ev, od

            a0, b0 = halves(0)
            m0 = ce[0] * a0
            m1 = ce[0] * b0
            for b in range(1, NB):
                ab, bb = halves(b)
                m0 = m0 + ce[b] * ab
                m1 = m1 + ce[b] * bb
            msg_v[k0, pl.ds(g2 * 32, 16)] = m0
            msg_v[k0, pl.ds(g2 * 32 + 16, 16)] = m1


def _edge_body(t_hbm, crep_hbm, src_hbm, dst_hbm, rel_hbm, z128_hbm,
               acc_out,
               acc_sh, src_sv, dst_sv, rel_sv, rows0, rows1,
               msg0, msg1, crep_v, gsem0, gsem1, asem0, asem1):
    c = lax.axis_index("c")
    s = lax.axis_index("s")
    wid = s * 2 + c

    pltpu.sync_copy(z128_hbm, acc_sh.at[pl.ds(s * STRIPE, STRIPE)])
    pltpu.sync_copy(crep_hbm, crep_v)
    plsc.subcore_barrier()

    base = wid * EDGES_PER_W

    def gather(co, buf, sem):
        return pltpu.async_copy(t_hbm.at[src_sv[pl.ds(co, 16)]], buf, sem)

    def gather_wait(buf, sem):
        pltpu.make_async_copy(t_hbm.at[src_sv[pl.ds(0, 16)]],
                              buf, sem).wait()

    def scatter(msg_b, co, sem):
        pltpu.async_copy(msg_b, acc_sh.at[dst_sv[pl.ds(co, 16)]], sem,
                         add=True)

    def scatter_wait(msg_b, sem):
        pltpu.make_async_copy(msg_b, acc_sh.at[dst_sv[pl.ds(0, 16)]],
                              sem).wait()

    def sup_body(sc, carry):
        soff = base + sc * SUP
        pltpu.sync_copy(src_hbm.at[pl.ds(soff, SUP)], src_sv)
        pltpu.sync_copy(dst_hbm.at[pl.ds(soff, SUP)], dst_sv)
        pltpu.sync_copy(rel_hbm.at[pl.ds(soff, SUP)], rel_sv)
        gather(0, rows0, gsem0)

        def pair_body(p, carry2):
            co = p * 32
            first = jnp.logical_and(sc == 0, p == 0)
            d1 = gather(co + 16, rows1, gsem1)
            gather_wait(rows0, gsem0)

            @pl.when(jnp.logical_not(first))
            def _():
                scatter_wait(msg0, asem0)

            _mix_chunk(rows0, 0, rel_sv, crep_v, msg0, co)
            scatter(msg0, co, asem0)

            @pl.when(p < NPAIR - 1)
            def _():
                gather(co + 32, rows0, gsem0)

            d1.wait()

            @pl.when(jnp.logical_not(first))
            def _():
                scatter_wait(msg1, asem1)

            _mix_chunk(rows1, 0, rel_sv, crep_v, msg1, co + 16)
            scatter(msg1, co + 16, asem1)
            return carry2

        lax.fori_loop(0, NPAIR, pair_body, 0)
        return carry

    lax.fori_loop(0, N_SUP, sup_body, 0)
    scatter_wait(msg0, asem0)
    scatter_wait(msg1, asem1)
    plsc.subcore_barrier()

    pltpu.sync_copy(acc_sh.at[pl.ds(s * STRIPE, STRIPE)],
                    acc_out.at[c, pl.ds(s * STRIPE, STRIPE)])


def _edge_pass(t_flat, crep, src_p, dst_p, rel_p, z128):
    mesh = plsc.VectorSubcoreMesh(core_axis_name="c", subcore_axis_name="s")
    fn = pl.kernel(
        _edge_body,
        mesh=mesh,
        out_type=jax.ShapeDtypeStruct((2, ACC_ROWS, H), jnp.float32),
        scratch_types=[
            pltpu.VMEM_SHARED((ACC_ROWS, H), jnp.float32),
            pltpu.VMEM((SUP,), jnp.int32),
            pltpu.VMEM((SUP,), jnp.int32),
            pltpu.VMEM((SUP,), jnp.int32),
            pltpu.VMEM((GC, 2, H), jnp.int32),
            pltpu.VMEM((GC, 2, H), jnp.int32),
            pltpu.VMEM((16, H), jnp.float32),
            pltpu.VMEM((16, H), jnp.float32),
            pltpu.VMEM((240 * CREP_W,), jnp.float32),
            pltpu.SemaphoreType.DMA,
            pltpu.SemaphoreType.DMA,
            pltpu.SemaphoreType.DMA,
            pltpu.SemaphoreType.DMA,
        ],
    )
    return fn(t_flat, crep, src_p, dst_p, rel_p, z128)


def _combine_basis_body(acc_ref, deg_ref, bias_ref, vlo_ref, vhi_ref,
                        out_ref):
    a = acc_ref[0] + acc_ref[1]
    d = deg_ref[0, :, 0:1] + deg_ref[1, :, 0:1]
    norm = jnp.where(d > 0, 1.0 / jnp.maximum(d, 1.0), 0.0)
    h = a * norm + bias_ref[:]
    lo = jnp.dot(h, vlo_ref[:], preferred_element_type=jnp.float32)
    hi = jnp.dot(h, vhi_ref[:], preferred_element_type=jnp.float32)
    out_ref[...] = _pack_i32(lo, hi)


def _combine_basis(acc, deg, bias, vlo, vhi):
    grid = (pl.cdiv(N_ENT, H),)
    w = NB * H // 2
    return pl.pallas_call(
        _combine_basis_body,
        grid=grid,
        in_specs=[
            pl.BlockSpec((2, H, H), lambda g: (0, g, 0)),
            pl.BlockSpec((2, H, H), lambda g: (0, g, 0)),
            pl.BlockSpec((1, H), lambda g: (0, 0)),
            pl.BlockSpec((H, w), lambda g: (0, 0)),
            pl.BlockSpec((H, w), lambda g: (0, 0)),
        ],
        out_specs=pl.BlockSpec((H, w), lambda g: (g, 0)),
        out_shape=jax.ShapeDtypeStruct((N_ENT, w), jnp.int32),
    )(acc, deg, bias, vlo, vhi)


def _final_body(acc_ref, deg_ref, bias_ref, out_ref):
    a = acc_ref[0] + acc_ref[1]
    d = deg_ref[0, :, 0:1] + deg_ref[1, :, 0:1]
    norm = jnp.where(d > 0, 1.0 / jnp.maximum(d, 1.0), 0.0)
    out_ref[...] = jnp.maximum(a * norm + bias_ref[:], 0.0)


def _final(acc, deg, bias):
    grid = (pl.cdiv(N_ENT, H),)
    return pl.pallas_call(
        _final_body,
        grid=grid,
        in_specs=[
            pl.BlockSpec((2, H, H), lambda g: (0, g, 0)),
            pl.BlockSpec((2, H, H), lambda g: (0, g, 0)),
            pl.BlockSpec((1, H), lambda g: (0, 0)),
        ],
        out_specs=pl.BlockSpec((H, H), lambda g: (g, 0)),
        out_shape=jax.ShapeDtypeStruct((N_ENT, H), jnp.float32),
    )(acc, deg, bias)


def kernel(ent_emb, rel_emb, V1, coeff1, bias1, V2, coeff2, bias2,
           W_ih0, W_hh0, b_ih0, b_hh0, W_ih1, W_hh1, b_ih1, b_hh1,
           h0, edge_index, rel_id):
    del rel_emb

    x_stack = ent_emb.reshape(3, NBATCH, H)
    def vsplit(v):
        vc = v.transpose(1, 0, 2).reshape(H, NB, NB, 2, 16)
        return (vc[:, :, :, 0, :].reshape(H, NB * H // 2),
                vc[:, :, :, 1, :].reshape(H, NB * H // 2))
    v1lo, v1hi = vsplit(V1)
    v2lo, v2hi = vsplit(V2)
    wi0 = W_ih0.T
    wh0 = W_hh0.T
    wi1 = W_ih1.T
    wh1 = W_hh1.T
    bi0 = b_ih0.reshape(1, 3 * H)
    bh0 = b_hh0.reshape(1, 3 * H)
    bi1 = b_ih1.reshape(1, 3 * H)
    bh1 = b_hh1.reshape(1, 3 * H)

    pad = E_PAD - E
    src_p = jnp.concatenate([edge_index[0], jnp.zeros((pad,), jnp.int32)])
    dst_p = jnp.concatenate(
        [edge_index[1], jnp.full((pad,), ACC_ROWS - 1, jnp.int32)])
    rel_p = jnp.concatenate([rel_id, jnp.zeros((pad,), jnp.int32)])
    z128 = jnp.zeros((STRIPE, H), jnp.float32)
    crep1 = jnp.pad(jnp.repeat(coeff1, 16, axis=1),
                    ((0, 240 - N_REL), (0, 0))).reshape(-1)
    crep2 = jnp.pad(jnp.repeat(coeff2, 16, axis=1),
                    ((0, 240 - N_REL), (0, 0))).reshape(-1)

    deg = _deg_pass(dst_p, z128)
    t1 = _gru_basis(x_stack, h0, wi0, wh0, bi0, bh0, wi1, wh1,
                    bi1, bh1, v1lo, v1hi).reshape(N_ENT, 2, H)

    acc1 = _edge_pass(t1, crep1, src_p, dst_p, rel_p, z128)

    t2 = _combine_basis(acc1, deg, bias1.reshape(1, H), v2lo, v2hi
                        ).reshape(N_ENT, 2, H)

    acc2 = _edge_pass(t2, crep2, src_p, dst_p, rel_p, z128)

    return _final(acc2, deg, bias2.reshape(1, H))
